# baseline (speedup 1.0000x reference)
import functools

import jax
import jax.numpy as jnp
import numpy as np
from jax import lax
from jax.experimental import pallas as pl
from jax.experimental.pallas import tpu as pltpu
from jax.experimental.pallas import tpu_sc as plsc

_GRID_SIZES = [0.01, 0.02, 0.04, 0.08, 0.16, 0.32, 0.64, 1.28]
_L = len(_GRID_SIZES)
_IN_DIM = 3
_HID = 32
_OUT = 128
_K = 27
_ROW = 128
_N = 16384
_BLK = 512
_SUB = 128
_GPB = 3
_B = _K * _N
_WPS = _B // _SUB // 32


def _offsets():
    o = np.array([[dx, dy, dz] for dx in (-1, 0, 1) for dy in (-1, 0, 1) for dz in (-1, 0, 1)], dtype=np.int64)
    return jnp.asarray(o)


def _build_nidx(p):
    b, n, _ = p.shape
    N = b * n
    offs = _offsets()
    batch_idx = jnp.repeat(jnp.arange(b, dtype=jnp.int64), n)
    out = []
    for g in _GRID_SIZES:
        gc = jnp.floor(p / g).astype(jnp.int64)
        gc = gc - gc.min(axis=1, keepdims=True)
        spatial = gc.max(axis=1).max(axis=0) + 1
        Sx, Sy, Sz = spatial[0], spatial[1], spatial[2]
        S_cap = int(np.floor(1.0 / g)) + 1
        flat = gc.reshape(-1, 3)
        keys = ((batch_idx * Sx + flat[:, 0]) * Sy + flat[:, 1]) * Sz + flat[:, 2]
        V = b * S_cap * S_cap * S_cap
        lookup = jnp.full((V,), -1, dtype=jnp.int32).at[keys].max(jnp.arange(N, dtype=jnp.int32))
        nb = flat[None, :, :] + offs[:, None, :]
        hi = spatial.astype(jnp.int64)
        inb = jnp.all((nb >= 0) & (nb < hi[None, None, :]), axis=-1)
        nbc = jnp.clip(nb, 0, hi[None, None, :] - 1)
        nkeys = ((batch_idx[None, :] * Sx + nbc[..., 0]) * Sy + nbc[..., 1]) * Sz + nbc[..., 2]
        if V > 0:
            nidx = lookup[nkeys]
            mask = inb & (nidx >= 0)
            base = (batch_idx.astype(jnp.int32) * (_N // 2))[None, :]
            nidxp = jnp.where(mask, nidx, base).astype(jnp.int32)
            c0 = (mask & (nidx == base)).astype(jnp.float32)
            out.append(('fine', nidxp.reshape(_B // _SUB // _GPB, _GPB, _SUB),
                        c0.T.reshape(_N, _K)))
        else:
            Vp = max(128, ((V + 127) // 128) * 128 + (128 if V % 128 == 0 else 0))
            rep = lookup
            R = (rep[:, None] == jnp.arange(N, dtype=jnp.int32)[None, :]).astype(jnp.float32)
            R = jnp.pad(R, ((0, Vp - V), (0, 0)))
            idxc = jnp.where(inb, nkeys, V).astype(jnp.int32)
            out.append(('coarse', idxc.reshape(1, _B), R))
    return out


def _pad_cols(v):
    return jnp.concatenate(
        [v, jnp.zeros((v.shape[0], _ROW - _HID), v.dtype)], axis=1)


def _zero_row0(i, v):
    rows = i * _BLK + lax.broadcasted_iota(jnp.int32, (_BLK, 1), 0)
    return jnp.where(lax.rem(rows, _N // 2) == 0, 0.0, v)


_HB = _N // 2 // _BLK


def _feats_body(p_ref, w1_ref, b1_ref, wl_ref, bl_ref, o_ref, r0_ref):
    i = pl.program_id(0)
    pts = jnp.dot(p_ref[...], w1_ref[...], preferred_element_type=jnp.float32) + b1_ref[...]
    for s in range(_L):
        f = jnp.dot(pts, wl_ref[s], preferred_element_type=jnp.float32) + bl_ref[s]
        o_ref[s] = _pad_cols(_zero_row0(i, f))

        @pl.when(i == 0)
        def _():
            r0_ref[s, 0] = _pad_cols(f[0:1])

        @pl.when(i == _HB)
        def _():
            r0_ref[s, 1] = _pad_cols(f[0:1])


def _feats_call(p2, W1, b1, Wl, bl):
    return pl.pallas_call(
        _feats_body,
        grid=(_N // _BLK,),
        in_specs=[
            pl.BlockSpec((_BLK, _IN_DIM), lambda i: (i, 0)),
            pl.BlockSpec((_IN_DIM, _HID), lambda i: (0, 0)),
            pl.BlockSpec((1, _HID), lambda i: (0, 0)),
            pl.BlockSpec((_L, _HID, _HID), lambda i: (0, 0, 0)),
            pl.BlockSpec((_L, 1, _HID), lambda i: (0, 0, 0)),
        ],
        out_specs=[
            pl.BlockSpec((_L, _BLK, _ROW), lambda i: (0, i, 0)),
            pl.BlockSpec((_L, 2, 1, _ROW), lambda i: (0, 0, 0, 0)),
        ],
        out_shape=[
            jax.ShapeDtypeStruct((_L, _N, _ROW), jnp.float32),
            jax.ShapeDtypeStruct((_L, 2, 1, _ROW), jnp.float32),
        ],
    )(p2, W1, b1[None, :], Wl, bl[:, None, :])


def _dot27(g_ref, w_ref):
    acc = jnp.dot(g_ref[0], w_ref[0], preferred_element_type=jnp.float32)
    for kk in range(1, _K):
        acc += jnp.dot(g_ref[kk], w_ref[kk], preferred_element_type=jnp.float32)
    return acc


def _corr(c0_ref, r0_ref, w_ref):
    m = jnp.concatenate(
        [jnp.dot(r0_ref[0], w_ref[kk], preferred_element_type=jnp.float32)
         for kk in range(_K)], axis=0)
    return jnp.dot(c0_ref[...], m, preferred_element_type=jnp.float32)


def _res_fix(r_ref, r0_ref):
    rows = pl.program_id(0) * _BLK + lax.broadcasted_iota(jnp.int32, (_BLK, 1), 0)
    return jnp.where(lax.rem(rows, _N // 2) == 0, r0_ref[0], r_ref[...])[:, :_HID]


def _conv1_body(g_ref, w_ref, b_ref, r_ref, rr0_ref, c0_ref, t0_ref, o_ref, o0_ref):
    i = pl.program_id(0)
    v = (_dot27(g_ref, w_ref) + b_ref[...] + _res_fix(r_ref, rr0_ref)
         + _corr(c0_ref, t0_ref, w_ref))
    o_ref[...] = _pad_cols(_zero_row0(i, v))

    @pl.when(i == 0)
    def _():
        o0_ref[0] = _pad_cols(v[0:1])

    @pl.when(i == _HB)
    def _():
        o0_ref[1] = _pad_cols(v[0:1])


def _conv1_call(G, Wf, bias, resid, resid_r0, C0, tbl_r0):
    return pl.pallas_call(
        _conv1_body,
        grid=(_N // _BLK,),
        in_specs=[
            pl.BlockSpec((_K, _BLK, _ROW), lambda i: (0, i, 0)),
            pl.BlockSpec((_K, _ROW, _HID), lambda i: (0, 0, 0)),
            pl.BlockSpec((1, _HID), lambda i: (0, 0)),
            pl.BlockSpec((_BLK, _ROW), lambda i: (i, 0)),
            pl.BlockSpec((1, 1, _ROW), lambda i: (i // _HB, 0, 0)),
            pl.BlockSpec((_BLK, _K), lambda i: (i, 0)),
            pl.BlockSpec((1, 1, _ROW), lambda i: (i // _HB, 0, 0)),
        ],
        out_specs=[
            pl.BlockSpec((_BLK, _ROW), lambda i: (i, 0)),
            pl.BlockSpec((2, 1, _ROW), lambda i: (0, 0, 0)),
        ],
        out_shape=[
            jax.ShapeDtypeStruct((_N, _ROW), jnp.float32),
            jax.ShapeDtypeStruct((2, 1, _ROW), jnp.float32),
        ],
    )(G, Wf, bias[None, :], resid, resid_r0, C0, tbl_r0)


def _conv2_body(g_ref, w_ref, b_ref, r1_ref, r10_ref, r2_ref, r20_ref,
                c0_ref, o_ref):
    o_ref[...] = (_dot27(g_ref, w_ref) + b_ref[...]
                  + _corr(c0_ref, r10_ref, w_ref)
                  + _res_fix(r1_ref, r10_ref) + _res_fix(r2_ref, r20_ref))


def _conv2_call(G, Wf, bias, f1, f1_r0, feats_i, feats_r0, C0):
    return pl.pallas_call(
        _conv2_body,
        grid=(_N // _BLK,),
        in_specs=[
            pl.BlockSpec((_K, _BLK, _ROW), lambda i: (0, i, 0)),
            pl.BlockSpec((_K, _ROW, _HID), lambda i: (0, 0, 0)),
            pl.BlockSpec((1, _HID), lambda i: (0, 0)),
            pl.BlockSpec((_BLK, _ROW), lambda i: (i, 0)),
            pl.BlockSpec((1, 1, _ROW), lambda i: (i // _HB, 0, 0)),
            pl.BlockSpec((_BLK, _ROW), lambda i: (i, 0)),
            pl.BlockSpec((1, 1, _ROW), lambda i: (i // _HB, 0, 0)),
            pl.BlockSpec((_BLK, _K), lambda i: (i, 0)),
        ],
        out_specs=pl.BlockSpec((_BLK, _HID), lambda i: (i, 0)),
        out_shape=jax.ShapeDtypeStruct((_N, _HID), jnp.float32),
    )(G, Wf, bias[None, :], f1, f1_r0, feats_i, feats_r0, C0)


def _hbuild_body(r_ref, f_ref, r0_ref, h_ref):
    rows = lax.broadcasted_iota(jnp.int32, (_N, 1), 0)
    f = f_ref[...]
    f = jnp.where(rows == 0, r0_ref[0], f)
    f = jnp.where(rows == _N // 2, r0_ref[1], f)
    h_ref[...] = jnp.dot(r_ref[...], f, preferred_element_type=jnp.float32)


def _hbuild_call(R, f_tbl, f_r0):
    Vp = R.shape[0]
    return pl.pallas_call(
        _hbuild_body,
        grid=(Vp // 128,),
        in_specs=[
            pl.BlockSpec((128, _N), lambda i: (i, 0)),
            pl.BlockSpec((_N, _ROW), lambda i: (0, 0)),
            pl.BlockSpec((2, 1, _ROW), lambda i: (0, 0, 0)),
        ],
        out_specs=pl.BlockSpec((128, _ROW), lambda i: (i, 0)),
        out_shape=jax.ShapeDtypeStruct((Vp, _ROW), jnp.float32),
    )(R, f_tbl, f_r0)


def _final_call(os_, W2, b2):
    def body(*refs):
        o_refs = refs[:_L]
        w_ref, b_ref, out_ref = refs[_L], refs[_L + 1], refs[_L + 2]
        x = jnp.concatenate([r[...] for r in o_refs], axis=1)
        out_ref[...] = jnp.dot(x, w_ref[...], preferred_element_type=jnp.float32) + b_ref[...]

    return pl.pallas_call(
        body,
        grid=(_N // _BLK,),
        in_specs=[pl.BlockSpec((_BLK, _HID), lambda i: (i, 0)) for _ in range(_L)]
        + [
            pl.BlockSpec((_L * _HID, _OUT), lambda i: (0, 0)),
            pl.BlockSpec((1, _OUT), lambda i: (0, 0)),
        ],
        out_specs=pl.BlockSpec((_BLK, _OUT), lambda i: (i, 0)),
        out_shape=jax.ShapeDtypeStruct((_N, _OUT), jnp.float32),
    )(*os_, W2, b2[None, :])


_VMESH = plsc.VectorSubcoreMesh(core_axis_name="c", subcore_axis_name="s")


@jax.jit
def _sc_gather(feats, nidx):
    @functools.partial(
        pl.kernel,
        out_type=jax.ShapeDtypeStruct((_B, _ROW), jnp.float32),
        mesh=_VMESH,
        scratch_types=[pltpu.SemaphoreType.DMA],
    )
    def k(x_hbm, i_hbm, o_hbm, sem):
        def body(i_vmem, o_vmem):
            cps = [pltpu.async_copy(x_hbm.at[i_vmem.at[0, j]],
                                    o_vmem.at[pl.ds(j * _SUB, _SUB)], sem)
                   for j in range(_GPB)]
            for cp in cps:
                cp.wait()

        pltpu.emit_pipeline(
            body,
            grid=(_B // _SUB // _GPB,),
            in_specs=[pl.BlockSpec((1, _GPB, _SUB), lambda i: (i, 0, 0))],
            out_specs=[pl.BlockSpec((_GPB * _SUB, _ROW), lambda i: (i, 0))],
            core_axis_name=("c", "s"),
            dimension_semantics=(pltpu.PARALLEL,),
        )(i_hbm, o_hbm)

    return k(feats, nidx).reshape(_K, _N, _ROW)


def kernel(p, params):
    b, n, _ = p.shape
    structs = _build_nidx(p)

    feats, feats_r0 = _feats_call(
        p.reshape(_N, _IN_DIM), params['W1'], params['b1'],
        params['Wl'], params['bl'])

    Wc = jnp.pad(params['Wc'], ((0, 0), (0, 0), (0, _ROW - _HID), (0, 0)))
    bc = params['bc']

    zeros_c0 = jnp.zeros((_N, _K), jnp.float32)
    os_ = []
    for i in range(_L):
        kind, idx_flat, aux = structs[i]
        feats_i = feats[i]
        fr0 = feats_r0[i]
        if kind == 'fine':
            C0 = aux
            G1 = _sc_gather(feats_i, idx_flat)
            f1, f1_r0 = _conv1_call(G1, Wc[2 * i], bc[2 * i],
                                    feats_i, fr0, C0, fr0)
            G2 = _sc_gather(f1, idx_flat)
        else:
            C0 = zeros_c0
            G1 = _sc_gather(_hbuild_call(aux, feats_i, fr0), idx_flat)
            f1, f1_r0 = _conv1_call(G1, Wc[2 * i], bc[2 * i],
                                    feats_i, fr0, C0, fr0)
            G2 = _sc_gather(_hbuild_call(aux, f1, f1_r0), idx_flat)
        o_i = _conv2_call(G2, Wc[2 * i + 1], bc[2 * i + 1],
                          f1, f1_r0, feats_i, fr0, C0)
        os_.append(o_i)

    out = _final_call(os_, params['W2'], params['b2'])
    return out.reshape(b, n, _OUT)


# speedup vs baseline: 2.3937x; 2.3937x over previous
import functools

import jax
import jax.numpy as jnp
import numpy as np
from jax import lax
from jax.experimental import pallas as pl
from jax.experimental.pallas import tpu as pltpu
from jax.experimental.pallas import tpu_sc as plsc

_GRID_SIZES = [0.01, 0.02, 0.04, 0.08, 0.16, 0.32, 0.64, 1.28]
_L = len(_GRID_SIZES)
_IN_DIM = 3
_HID = 32
_OUT = 128
_K = 27
_ROW = 128
_N = 16384
_BLK = 512
_SUB = 128
_GPB = 3
_B = _K * _N
_WPS = _B // _SUB // 32


def _offsets():
    o = np.array([[dx, dy, dz] for dx in (-1, 0, 1) for dy in (-1, 0, 1) for dz in (-1, 0, 1)], dtype=np.int64)
    return jnp.asarray(o)


def _build_nidx(p):
    b, n, _ = p.shape
    N = b * n
    offs = _offsets()
    batch_idx = jnp.repeat(jnp.arange(b, dtype=jnp.int64), n)
    out = []
    for g in _GRID_SIZES:
        gc = jnp.floor(p / g).astype(jnp.int64)
        gc = gc - gc.min(axis=1, keepdims=True)
        spatial = gc.max(axis=1).max(axis=0) + 1
        Sx, Sy, Sz = spatial[0], spatial[1], spatial[2]
        S_cap = int(np.floor(1.0 / g)) + 1
        flat = gc.reshape(-1, 3)
        keys = ((batch_idx * Sx + flat[:, 0]) * Sy + flat[:, 1]) * Sz + flat[:, 2]
        V = b * S_cap * S_cap * S_cap
        lookup = jnp.full((V,), -1, dtype=jnp.int32).at[keys].max(jnp.arange(N, dtype=jnp.int32))
        nb = flat[None, :, :] + offs[:, None, :]
        hi = spatial.astype(jnp.int64)
        inb = jnp.all((nb >= 0) & (nb < hi[None, None, :]), axis=-1)
        nbc = jnp.clip(nb, 0, hi[None, None, :] - 1)
        nkeys = ((batch_idx[None, :] * Sx + nbc[..., 0]) * Sy + nbc[..., 1]) * Sz + nbc[..., 2]
        if V > 0:
            nidx = lookup[nkeys]
            mask = inb & (nidx >= 0)
            base = (batch_idx.astype(jnp.int32) * (_N // 2))[None, :]
            nidxp = jnp.where(mask, nidx, base).astype(jnp.int32)
            c0 = (mask & (nidx == base)).astype(jnp.float32)
            out.append(('fine', nidxp.reshape(1, _B), c0.T.reshape(_N, _K)))
        else:
            Vp = max(128, ((V + 127) // 128) * 128 + (128 if V % 128 == 0 else 0))
            rep = lookup
            R = (rep[:, None] == jnp.arange(N, dtype=jnp.int32)[None, :]).astype(jnp.float32)
            R = jnp.pad(R, ((0, Vp - V), (0, 0)))
            idxc = jnp.where(inb, nkeys, V).astype(jnp.int32)
            out.append(('coarse', idxc.reshape(1, _B), R))
    return out


def _pad_cols(v):
    return jnp.concatenate(
        [v, jnp.zeros((v.shape[0], _ROW - _HID), v.dtype)], axis=1)


def _zero_row0(i, v):
    rows = i * _BLK + lax.broadcasted_iota(jnp.int32, (_BLK, 1), 0)
    return jnp.where(lax.rem(rows, _N // 2) == 0, 0.0, v)


_HB = _N // 2 // _BLK


def _feats_body(p_ref, w1_ref, b1_ref, wl_ref, bl_ref, o_ref, r0_ref):
    i = pl.program_id(0)
    pts = jnp.dot(p_ref[...], w1_ref[...], preferred_element_type=jnp.float32) + b1_ref[...]
    for s in range(_L):
        f = jnp.dot(pts, wl_ref[s], preferred_element_type=jnp.float32) + bl_ref[s]
        o_ref[s] = _pad_cols(_zero_row0(i, f))

        @pl.when(i == 0)
        def _():
            r0_ref[s, 0] = _pad_cols(f[0:1])

        @pl.when(i == _HB)
        def _():
            r0_ref[s, 1] = _pad_cols(f[0:1])


def _feats_call(p2, W1, b1, Wl, bl):
    return pl.pallas_call(
        _feats_body,
        grid=(_N // _BLK,),
        in_specs=[
            pl.BlockSpec((_BLK, _IN_DIM), lambda i: (i, 0)),
            pl.BlockSpec((_IN_DIM, _HID), lambda i: (0, 0)),
            pl.BlockSpec((1, _HID), lambda i: (0, 0)),
            pl.BlockSpec((_L, _HID, _HID), lambda i: (0, 0, 0)),
            pl.BlockSpec((_L, 1, _HID), lambda i: (0, 0, 0)),
        ],
        out_specs=[
            pl.BlockSpec((_L, _BLK, _ROW), lambda i: (0, i, 0)),
            pl.BlockSpec((_L, 2, 1, _ROW), lambda i: (0, 0, 0, 0)),
        ],
        out_shape=[
            jax.ShapeDtypeStruct((_L, _N, _ROW), jnp.float32),
            jax.ShapeDtypeStruct((_L, 2, 1, _ROW), jnp.float32),
        ],
    )(p2, W1, b1[None, :], Wl, bl[:, None, :])


def _dot27(g_ref, w_ref):
    acc = jnp.dot(g_ref[0], w_ref[0], preferred_element_type=jnp.float32)
    for kk in range(1, _K):
        acc += jnp.dot(g_ref[kk], w_ref[kk], preferred_element_type=jnp.float32)
    return acc


def _corr(c0_ref, r0_ref, w_ref):
    m = jnp.concatenate(
        [jnp.dot(r0_ref[0], w_ref[kk], preferred_element_type=jnp.float32)
         for kk in range(_K)], axis=0)
    return jnp.dot(c0_ref[...], m, preferred_element_type=jnp.float32)


def _res_fix(r_ref, r0_ref):
    rows = pl.program_id(0) * _BLK + lax.broadcasted_iota(jnp.int32, (_BLK, 1), 0)
    return jnp.where(lax.rem(rows, _N // 2) == 0, r0_ref[0], r_ref[...])[:, :_HID]


def _conv1_body(g_ref, w_ref, b_ref, r_ref, rr0_ref, c0_ref, t0_ref, o_ref, o0_ref):
    i = pl.program_id(0)
    v = (_dot27(g_ref, w_ref) + b_ref[...] + _res_fix(r_ref, rr0_ref)
         + _corr(c0_ref, t0_ref, w_ref))
    o_ref[...] = _pad_cols(_zero_row0(i, v))

    @pl.when(i == 0)
    def _():
        o0_ref[0] = _pad_cols(v[0:1])

    @pl.when(i == _HB)
    def _():
        o0_ref[1] = _pad_cols(v[0:1])


def _conv1_call(G, Wf, bias, resid, resid_r0, C0, tbl_r0):
    return pl.pallas_call(
        _conv1_body,
        grid=(_N // _BLK,),
        in_specs=[
            pl.BlockSpec((_K, _BLK, _ROW), lambda i: (0, i, 0)),
            pl.BlockSpec((_K, _ROW, _HID), lambda i: (0, 0, 0)),
            pl.BlockSpec((1, _HID), lambda i: (0, 0)),
            pl.BlockSpec((_BLK, _ROW), lambda i: (i, 0)),
            pl.BlockSpec((1, 1, _ROW), lambda i: (i // _HB, 0, 0)),
            pl.BlockSpec((_BLK, _K), lambda i: (i, 0)),
            pl.BlockSpec((1, 1, _ROW), lambda i: (i // _HB, 0, 0)),
        ],
        out_specs=[
            pl.BlockSpec((_BLK, _ROW), lambda i: (i, 0)),
            pl.BlockSpec((2, 1, _ROW), lambda i: (0, 0, 0)),
        ],
        out_shape=[
            jax.ShapeDtypeStruct((_N, _ROW), jnp.float32),
            jax.ShapeDtypeStruct((2, 1, _ROW), jnp.float32),
        ],
    )(G, Wf, bias[None, :], resid, resid_r0, C0, tbl_r0)


def _conv2_body(g_ref, w_ref, b_ref, r1_ref, r10_ref, r2_ref, r20_ref,
                c0_ref, o_ref):
    o_ref[...] = (_dot27(g_ref, w_ref) + b_ref[...]
                  + _corr(c0_ref, r10_ref, w_ref)
                  + _res_fix(r1_ref, r10_ref) + _res_fix(r2_ref, r20_ref))


def _conv2_call(G, Wf, bias, f1, f1_r0, feats_i, feats_r0, C0):
    return pl.pallas_call(
        _conv2_body,
        grid=(_N // _BLK,),
        in_specs=[
            pl.BlockSpec((_K, _BLK, _ROW), lambda i: (0, i, 0)),
            pl.BlockSpec((_K, _ROW, _HID), lambda i: (0, 0, 0)),
            pl.BlockSpec((1, _HID), lambda i: (0, 0)),
            pl.BlockSpec((_BLK, _ROW), lambda i: (i, 0)),
            pl.BlockSpec((1, 1, _ROW), lambda i: (i // _HB, 0, 0)),
            pl.BlockSpec((_BLK, _ROW), lambda i: (i, 0)),
            pl.BlockSpec((1, 1, _ROW), lambda i: (i // _HB, 0, 0)),
            pl.BlockSpec((_BLK, _K), lambda i: (i, 0)),
        ],
        out_specs=pl.BlockSpec((_BLK, _HID), lambda i: (i, 0)),
        out_shape=jax.ShapeDtypeStruct((_N, _HID), jnp.float32),
    )(G, Wf, bias[None, :], f1, f1_r0, feats_i, feats_r0, C0)


def _hbuild_body(r_ref, f_ref, r0_ref, h_ref):
    rows = lax.broadcasted_iota(jnp.int32, (_N, 1), 0)
    f = f_ref[...]
    f = jnp.where(rows == 0, r0_ref[0], f)
    f = jnp.where(rows == _N // 2, r0_ref[1], f)
    h_ref[...] = jnp.dot(r_ref[...], f, preferred_element_type=jnp.float32)


def _hbuild_call(R, f_tbl, f_r0):
    Vp = R.shape[0]
    return pl.pallas_call(
        _hbuild_body,
        grid=(Vp // 128,),
        in_specs=[
            pl.BlockSpec((128, _N), lambda i: (i, 0)),
            pl.BlockSpec((_N, _ROW), lambda i: (0, 0)),
            pl.BlockSpec((2, 1, _ROW), lambda i: (0, 0, 0)),
        ],
        out_specs=pl.BlockSpec((128, _ROW), lambda i: (i, 0)),
        out_shape=jax.ShapeDtypeStruct((Vp, _ROW), jnp.float32),
    )(R, f_tbl, f_r0)


def _final_call(os_, W2, b2):
    def body(*refs):
        o_refs = refs[:_L]
        w_ref, b_ref, out_ref = refs[_L], refs[_L + 1], refs[_L + 2]
        x = jnp.concatenate([r[...] for r in o_refs], axis=1)
        out_ref[...] = jnp.dot(x, w_ref[...], preferred_element_type=jnp.float32) + b_ref[...]

    return pl.pallas_call(
        body,
        grid=(_N // _BLK,),
        in_specs=[pl.BlockSpec((_BLK, _HID), lambda i: (i, 0)) for _ in range(_L)]
        + [
            pl.BlockSpec((_L * _HID, _OUT), lambda i: (0, 0)),
            pl.BlockSpec((1, _OUT), lambda i: (0, 0)),
        ],
        out_specs=pl.BlockSpec((_BLK, _OUT), lambda i: (i, 0)),
        out_shape=jax.ShapeDtypeStruct((_N, _OUT), jnp.float32),
    )(*os_, W2, b2[None, :])


_VMESH = plsc.VectorSubcoreMesh(core_axis_name="c", subcore_axis_name="s")


@jax.jit
def _sc_gather(feats, nidx):
    @functools.partial(
        pl.kernel,
        out_type=jax.ShapeDtypeStruct((_B, _ROW), jnp.float32),
        mesh=_VMESH,
    )
    def k(x_hbm, i_hbm, o_hbm):
        def body(i_vmem, o_vmem):
            pltpu.sync_copy(x_hbm.at[i_vmem.at[0]], o_vmem)

        pltpu.emit_pipeline(
            body,
            grid=(_B // _SUB,),
            in_specs=[pl.BlockSpec((1, _SUB), lambda i: (0, i))],
            out_specs=[pl.BlockSpec((_SUB, _ROW), lambda i: (i, 0))],
            core_axis_name=("c", "s"),
            dimension_semantics=(pltpu.PARALLEL,),
        )(i_hbm, o_hbm)

    return k(feats, nidx).reshape(_K, _N, _ROW)


def kernel(p, params):
    b, n, _ = p.shape
    structs = _build_nidx(p)

    feats, feats_r0 = _feats_call(
        p.reshape(_N, _IN_DIM), params['W1'], params['b1'],
        params['Wl'], params['bl'])

    Wc = jnp.pad(params['Wc'], ((0, 0), (0, 0), (0, _ROW - _HID), (0, 0)))
    bc = params['bc']

    zeros_c0 = jnp.zeros((_N, _K), jnp.float32)
    os_ = []
    for i in range(_L):
        kind, idx_flat, aux = structs[i]
        feats_i = feats[i]
        fr0 = feats_r0[i]
        if kind == 'fine':
            C0 = aux
            G1 = _sc_gather(feats_i, idx_flat)
            f1, f1_r0 = _conv1_call(G1, Wc[2 * i], bc[2 * i],
                                    feats_i, fr0, C0, fr0)
            G2 = _sc_gather(f1, idx_flat)
        else:
            C0 = zeros_c0
            G1 = _sc_gather(_hbuild_call(aux, feats_i, fr0), idx_flat)
            f1, f1_r0 = _conv1_call(G1, Wc[2 * i], bc[2 * i],
                                    feats_i, fr0, C0, fr0)
            G2 = _sc_gather(_hbuild_call(aux, f1, f1_r0), idx_flat)
        o_i = _conv2_call(G2, Wc[2 * i + 1], bc[2 * i + 1],
                          f1, f1_r0, feats_i, fr0, C0)
        os_.append(o_i)

    out = _final_call(os_, params['W2'], params['b2'])
    return out.reshape(b, n, _OUT)
